# Initial kernel scaffold; baseline (speedup 1.0000x reference)
import jax
import jax.numpy as jnp
from jax.experimental import pallas as pl


def kernel(batch, lattice, pos, edge_index, edge_cell_shift, triplet_edge_index):
    raise NotImplementedError("write your pallas kernel here")


# speedup vs baseline: 38.6367x; 38.6367x over previous
import functools

import jax
import jax.numpy as jnp
from jax import lax
from jax.experimental import pallas as pl
from jax.experimental.pallas import tpu as pltpu
from jax.experimental.pallas import tpu_sc as plsc

_L = 16
_CHUNK = 80
_RW = 16


def _rsqrt(n2):
  xhalf = n2 * 0.5
  i = plsc.bitcast(n2, jnp.int32)
  i = jnp.int32(0x5F3759DF) - lax.shift_right_logical(i, 1)
  y = plsc.bitcast(i, jnp.float32)
  y = y * (1.5 - xhalf * y * y)
  y = y * (1.5 - xhalf * y * y)
  y = y * (1.5 - xhalf * y * y)
  return y


def _gather_rows(table_hbm, idx2d_v, dst_v, sem, nch):
  copies = []
  for j in range(nch):
    copies.append(pltpu.async_copy(
        table_hbm.at[idx2d_v.at[j]],
        dst_v.at[pl.ds(j * _CHUNK, _CHUNK)], sem))
  return copies


def _edge_body(e_per_w, blk, nc, half_rows, ng1,
               posb_hbm, lat_hbm, eidx2d_hbm, ecs_hbm,
               dist_hbm, pvd_hbm,
               lat_v, idx0_v, idx1_v, ecs_v, rowsi_v, rowsj_v,
               pvd_v, dist_v, sem):
  wid = lax.axis_index("s") * nc + lax.axis_index("c")
  base_w = wid * e_per_w
  nch = blk // _CHUNK

  pltpu.sync_copy(lat_hbm, lat_v)

  lanes = lax.iota(jnp.int32, _L)
  c0 = jnp.zeros((_L,), jnp.int32)
  c1 = jnp.full((_L,), 1, jnp.int32)
  c2 = jnp.full((_L,), 2, jnp.int32)
  c3 = jnp.full((_L,), 3, jnp.int32)

  def block(b, _):
    base = base_w + b * blk
    rbase = base // _CHUNK
    pltpu.sync_copy(eidx2d_hbm.at[pl.ds(rbase, nch)], idx0_v)
    pltpu.sync_copy(eidx2d_hbm.at[pl.ds(half_rows + rbase, nch)], idx1_v)
    pltpu.sync_copy(ecs_hbm.at[pl.ds(base, blk)], ecs_v)
    copies = _gather_rows(posb_hbm, idx0_v, rowsi_v, sem, nch)
    copies += _gather_rows(posb_hbm, idx1_v, rowsj_v, sem, nch)
    for c in copies:
      c.wait()

    def group(g, _):
      off = g * _L
      idx = lanes + off
      xi = plsc.load_gather(rowsi_v, [idx, c0])
      yi = plsc.load_gather(rowsi_v, [idx, c1])
      zi = plsc.load_gather(rowsi_v, [idx, c2])
      bf = plsc.load_gather(rowsi_v, [idx, c3])
      bi = bf.astype(jnp.int32)
      xj = plsc.load_gather(rowsj_v, [idx, c0])
      yj = plsc.load_gather(rowsj_v, [idx, c1])
      zj = plsc.load_gather(rowsj_v, [idx, c2])
      e0 = plsc.load_gather(ecs_v, [idx, c0]).astype(jnp.float32)
      e1 = plsc.load_gather(ecs_v, [idx, c1]).astype(jnp.float32)
      e2 = plsc.load_gather(ecs_v, [idx, c2]).astype(jnp.float32)
      b9 = jnp.minimum(jnp.maximum(bi, 0), ng1) * 9
      l00 = plsc.load_gather(lat_v, [b9])
      l01 = plsc.load_gather(lat_v, [b9 + 1])
      l02 = plsc.load_gather(lat_v, [b9 + 2])
      l10 = plsc.load_gather(lat_v, [b9 + 3])
      l11 = plsc.load_gather(lat_v, [b9 + 4])
      l12 = plsc.load_gather(lat_v, [b9 + 5])
      l20 = plsc.load_gather(lat_v, [b9 + 6])
      l21 = plsc.load_gather(lat_v, [b9 + 7])
      l22 = plsc.load_gather(lat_v, [b9 + 8])
      sx = e0 * l00 + e1 * l10 + e2 * l20
      sy = e0 * l01 + e1 * l11 + e2 * l21
      sz = e0 * l02 + e1 * l12 + e2 * l22
      px = xj + sx - xi
      py = yj + sy - yi
      pz = zj + sz - zi
      n2 = px * px + py * py + pz * pz
      d = n2 * _rsqrt(n2)
      plsc.store_scatter(pvd_v, [idx, c0], px)
      plsc.store_scatter(pvd_v, [idx, c1], py)
      plsc.store_scatter(pvd_v, [idx, c2], pz)
      plsc.store_scatter(pvd_v, [idx, c3], d)
      dist_v[pl.ds(off, _L)] = d
      return ()

    lax.fori_loop(0, blk // _L, group, (), unroll=False)
    pltpu.sync_copy(dist_v, dist_hbm.at[pl.ds(base, blk)])
    pltpu.sync_copy(pvd_v, pvd_hbm.at[pl.ds(base, blk)])
    return ()

  lax.fori_loop(0, e_per_w // blk, block, (), unroll=False)


def _tri_body(t_per_w, blk, nc, half_rows,
              pvd_hbm, tidx2d_hbm, ang_hbm,
              t0_v, t1_v, rows0_v, rows1_v, ang_v, sem):
  wid = lax.axis_index("s") * nc + lax.axis_index("c")
  base_w = wid * t_per_w
  nch = blk // _CHUNK

  lanes = lax.iota(jnp.int32, _L)
  c0 = jnp.zeros((_L,), jnp.int32)
  c1 = jnp.full((_L,), 1, jnp.int32)
  c2 = jnp.full((_L,), 2, jnp.int32)
  c3 = jnp.full((_L,), 3, jnp.int32)

  def block(b, _):
    base = base_w + b * blk
    rbase = base // _CHUNK
    pltpu.sync_copy(tidx2d_hbm.at[pl.ds(rbase, nch)], t0_v)
    pltpu.sync_copy(tidx2d_hbm.at[pl.ds(half_rows + rbase, nch)], t1_v)
    copies = _gather_rows(pvd_hbm, t0_v, rows0_v, sem, nch)
    copies += _gather_rows(pvd_hbm, t1_v, rows1_v, sem, nch)
    for c in copies:
      c.wait()

    def group(g, _):
      off = g * _L
      idx = lanes + off
      x0 = plsc.load_gather(rows0_v, [idx, c0])
      y0 = plsc.load_gather(rows0_v, [idx, c1])
      z0 = plsc.load_gather(rows0_v, [idx, c2])
      d0 = plsc.load_gather(rows0_v, [idx, c3])
      x1 = plsc.load_gather(rows1_v, [idx, c0])
      y1 = plsc.load_gather(rows1_v, [idx, c1])
      z1 = plsc.load_gather(rows1_v, [idx, c2])
      d1 = plsc.load_gather(rows1_v, [idx, c3])
      cos = (x0 * x1 + y0 * y1 + z0 * z1) / (d0 * d1)
      cos = jnp.minimum(jnp.maximum(cos, -1.0), 1.0)
      ang_v[pl.ds(off, _L)] = cos
      return ()

    lax.fori_loop(0, blk // _L, group, (), unroll=False)
    pltpu.sync_copy(ang_v, ang_hbm.at[pl.ds(base, blk)])
    return ()

  lax.fori_loop(0, t_per_w // blk, block, (), unroll=False)


def kernel(batch, lattice, pos, edge_index, edge_cell_shift,
           triplet_edge_index):
  n_graphs = lattice.shape[0]
  n_nodes = pos.shape[0]
  n_edges = edge_index.shape[1]
  n_tri = triplet_edge_index.shape[1]

  info = plsc.get_sparse_core_info()
  nw = info.num_cores * info.num_subcores
  mesh = plsc.VectorSubcoreMesh(core_axis_name="c", subcore_axis_name="s")

  e_per_w = n_edges // nw
  t_per_w = n_tri // nw
  eblk = 2000
  tblk = 2000

  posb = jnp.zeros((n_nodes, _RW), jnp.float32)
  posb = posb.at[:, :3].set(pos)
  posb = posb.at[:, 3].set(batch.astype(jnp.float32))
  lat_flat = lattice.reshape(n_graphs * 9)
  eidx2d = edge_index.reshape(2 * n_edges // _CHUNK, _CHUNK)
  tidx2d = triplet_edge_index.reshape(2 * n_tri // _CHUNK, _CHUNK)

  edge_k = pl.kernel(
      functools.partial(_edge_body, e_per_w, eblk, info.num_cores,
                        n_edges // _CHUNK, n_graphs - 1),
      out_type=(
          jax.ShapeDtypeStruct((n_edges,), jnp.float32),
          jax.ShapeDtypeStruct((n_edges, _RW), jnp.float32),
      ),
      mesh=mesh,
      compiler_params=pltpu.CompilerParams(
          needs_layout_passes=False, use_tc_tiling_on_sc=False),
      scratch_types=[
          pltpu.VMEM((n_graphs * 9,), jnp.float32),
          pltpu.VMEM((eblk // _CHUNK, _CHUNK), jnp.int32),
          pltpu.VMEM((eblk // _CHUNK, _CHUNK), jnp.int32),
          pltpu.VMEM((eblk, 3), jnp.int32),
          pltpu.VMEM((eblk, _RW), jnp.float32),
          pltpu.VMEM((eblk, _RW), jnp.float32),
          pltpu.VMEM((eblk, _RW), jnp.float32),
          pltpu.VMEM((eblk,), jnp.float32),
          pltpu.SemaphoreType.DMA,
      ],
  )
  dist, pvd = edge_k(posb, lat_flat, eidx2d, edge_cell_shift)

  tri_k = pl.kernel(
      functools.partial(_tri_body, t_per_w, tblk, info.num_cores,
                        n_tri // _CHUNK),
      out_type=jax.ShapeDtypeStruct((n_tri,), jnp.float32),
      mesh=mesh,
      compiler_params=pltpu.CompilerParams(
          needs_layout_passes=False, use_tc_tiling_on_sc=False),
      scratch_types=[
          pltpu.VMEM((tblk // _CHUNK, _CHUNK), jnp.int32),
          pltpu.VMEM((tblk // _CHUNK, _CHUNK), jnp.int32),
          pltpu.VMEM((tblk, _RW), jnp.float32),
          pltpu.VMEM((tblk, _RW), jnp.float32),
          pltpu.VMEM((tblk,), jnp.float32),
          pltpu.SemaphoreType.DMA,
      ],
  )
  angles = tri_k(pvd, tidx2d)
  return dist, angles
